# Initial kernel scaffold; baseline (speedup 1.0000x reference)
import functools

import jax
import jax.numpy as jnp
from jax import lax
from jax.experimental import pallas as pl
from jax.experimental.pallas import tpu as pltpu
from jax.experimental.pallas import tpu_sc as plsc

LSEQ = 200
EMBD = 32
BATCH = 4096
NWORK = 32
BPW = BATCH // NWORK
HALF = 16


def _sc_pool(iit, emb):
    mesh = plsc.VectorSubcoreMesh(core_axis_name="c", subcore_axis_name="s")

    @functools.partial(
        pl.kernel,
        out_type=jax.ShapeDtypeStruct((BATCH, EMBD), jnp.float32),
        mesh=mesh,
        scratch_types=[
            pltpu.VMEM((BPW, LSEQ), jnp.int32),
            pltpu.VMEM((LSEQ, EMBD), jnp.float32),
            pltpu.VMEM((LSEQ, EMBD), jnp.float32),
            pltpu.VMEM((BPW, EMBD), jnp.float32),
            pltpu.SemaphoreType.DMA,
            pltpu.SemaphoreType.DMA,
        ],
    )
    def pool(ii_hbm, emb_hbm, out_hbm, idx_v, rows0, rows1, xm_v, sem0, sem1):
        wid = lax.axis_index("s") * 2 + lax.axis_index("c")
        base = wid * BPW
        pltpu.sync_copy(ii_hbm.at[pl.ds(base, BPW), :], idx_v)

        def start_gather(b, rows_ref, sem):
            pltpu.async_copy(emb_hbm.at[idx_v.at[b]], rows_ref, sem)

        def wait_gather(b, rows_ref, sem):
            pltpu.make_async_copy(emb_hbm.at[idx_v.at[b]], rows_ref, sem).wait()

        def reduce_rows(rows_ref, b):
            zero = jnp.zeros((HALF,), jnp.float32)

            def body(l, carry):
                s0, s1, c0, c1 = carry
                x0 = rows_ref[l, pl.ds(0, HALF)]
                x1 = rows_ref[l, pl.ds(HALF, HALF)]
                s0 = s0 + x0
                s1 = s1 + x1
                c0 = c0 + (x0 != 0.0).astype(jnp.float32)
                c1 = c1 + (x1 != 0.0).astype(jnp.float32)
                return s0, s1, c0, c1

            s0, s1, c0, c1 = lax.fori_loop(0, LSEQ, body, (zero, zero, zero, zero))
            one = jnp.full((HALF,), 1.0, jnp.float32)
            xm_v[b, pl.ds(0, HALF)] = s0 / jnp.maximum(c0, one)
            xm_v[b, pl.ds(HALF, HALF)] = s1 / jnp.maximum(c1, one)

        start_gather(0, rows0, sem0)

        def outer(j, carry):
            b0 = 2 * j
            start_gather(b0 + 1, rows1, sem1)
            wait_gather(b0, rows0, sem0)
            reduce_rows(rows0, b0)

            @pl.when(j < BPW // 2 - 1)
            def _():
                start_gather(b0 + 2, rows0, sem0)

            wait_gather(b0 + 1, rows1, sem1)
            reduce_rows(rows1, b0 + 1)
            return carry

        lax.fori_loop(0, BPW // 2, outer, 0)
        pltpu.sync_copy(xm_v, out_hbm.at[pl.ds(base, BPW), :])

    return pool(iit, emb)


def _tc_mlp(xm, w1, b1, w2, b2):
    h1 = w1.shape[1]
    h2 = w2.shape[1]
    bt = 512

    def body(x_ref, w1_ref, b1_ref, w2_ref, b2_ref, o_ref):
        x = x_ref[...]
        h = jnp.dot(x, w1_ref[...], preferred_element_type=jnp.float32)
        h = jnp.maximum(h + b1_ref[...], 0.0)
        z = jnp.dot(h, w2_ref[...], preferred_element_type=jnp.float32)
        o_ref[...] = jax.nn.sigmoid(z + b2_ref[...])

    return pl.pallas_call(
        body,
        grid=(BATCH // bt,),
        in_specs=[
            pl.BlockSpec((bt, EMBD), lambda i: (i, 0)),
            pl.BlockSpec((EMBD, h1), lambda i: (0, 0)),
            pl.BlockSpec((1, h1), lambda i: (0, 0)),
            pl.BlockSpec((h1, h2), lambda i: (0, 0)),
            pl.BlockSpec((1, h2), lambda i: (0, 0)),
        ],
        out_specs=pl.BlockSpec((bt, h2), lambda i: (i, 0)),
        out_shape=jax.ShapeDtypeStruct((BATCH, h2), jnp.float32),
    )(xm, w1, b1.reshape(1, -1), w2, b2.reshape(1, -1))


def kernel(II, emb, W1, b1, W2, b2):
    iit = II.T
    xm = _sc_pool(iit, emb)
    return _tc_mlp(xm, W1, b1, W2, b2)


# speedup vs baseline: 1.9687x; 1.9687x over previous
import functools

import jax
import jax.numpy as jnp
from jax import lax
from jax.experimental import pallas as pl
from jax.experimental.pallas import tpu as pltpu
from jax.experimental.pallas import tpu_sc as plsc

LSEQ = 200
EMBD = 32
BATCH = 4096
NWORK = 32
BPW = BATCH // NWORK
HALF = 16


def _sc_pool(iit, emb):
    mesh = plsc.VectorSubcoreMesh(core_axis_name="c", subcore_axis_name="s")

    @functools.partial(
        pl.kernel,
        out_type=jax.ShapeDtypeStruct((BATCH, EMBD), jnp.float32),
        mesh=mesh,
        scratch_types=[
            pltpu.VMEM((BPW * LSEQ,), jnp.int32),
            pltpu.VMEM((LSEQ, EMBD), jnp.float32),
            pltpu.VMEM((LSEQ, EMBD), jnp.float32),
            pltpu.VMEM((BPW, EMBD), jnp.float32),
            pltpu.SemaphoreType.DMA,
            pltpu.SemaphoreType.DMA,
        ],
        compiler_params=pltpu.CompilerParams(use_tc_tiling_on_sc=False),
    )
    def pool(ii_hbm, emb_hbm, out_hbm, idx_v, rows0, rows1, xm_v, sem0, sem1):
        wid = lax.axis_index("s") * 2 + lax.axis_index("c")
        base = wid * BPW
        pltpu.sync_copy(ii_hbm.at[pl.ds(base * LSEQ, BPW * LSEQ)], idx_v)

        def idx_slice(b):
            return idx_v.at[pl.ds(pl.multiple_of(b * LSEQ, 8), LSEQ)]

        def start_gather(b, rows_ref, sem):
            pltpu.async_copy(emb_hbm.at[idx_slice(b)], rows_ref, sem)

        def wait_gather(b, rows_ref, sem):
            pltpu.make_async_copy(emb_hbm.at[idx_slice(b)], rows_ref, sem).wait()

        def reduce_rows(rows_ref, b):
            zero = jnp.zeros((HALF,), jnp.float32)
            one = jnp.full((HALF,), 1.0, jnp.float32)

            def body(l, carry):
                s0, s1, c0, c1 = carry
                x0 = rows_ref[l, pl.ds(0, HALF)]
                x1 = rows_ref[l, pl.ds(HALF, HALF)]
                s0 = s0 + x0
                s1 = s1 + x1
                c0 = c0 + jnp.where(x0 != zero, one, zero)
                c1 = c1 + jnp.where(x1 != zero, one, zero)
                return s0, s1, c0, c1

            s0, s1, c0, c1 = lax.fori_loop(0, LSEQ, body, (zero, zero, zero, zero))
            xm_v[b, pl.ds(0, HALF)] = s0 / jnp.maximum(c0, one)
            xm_v[b, pl.ds(HALF, HALF)] = s1 / jnp.maximum(c1, one)

        start_gather(0, rows0, sem0)

        def outer(j, carry):
            b0 = 2 * j
            start_gather(b0 + 1, rows1, sem1)
            wait_gather(b0, rows0, sem0)
            reduce_rows(rows0, b0)

            @pl.when(j < BPW // 2 - 1)
            def _():
                start_gather(b0 + 2, rows0, sem0)

            wait_gather(b0 + 1, rows1, sem1)
            reduce_rows(rows1, b0 + 1)
            return carry

        lax.fori_loop(0, BPW // 2, outer, 0)
        pltpu.sync_copy(xm_v, out_hbm.at[pl.ds(base, BPW), :])

    return pool(iit, emb)


def _tc_mlp(xm, w1, b1, w2, b2):
    h1 = w1.shape[1]
    h2 = w2.shape[1]
    bt = 512

    def body(x_ref, w1_ref, b1_ref, w2_ref, b2_ref, o_ref):
        x = x_ref[...]
        h = jnp.dot(x, w1_ref[...], preferred_element_type=jnp.float32)
        h = jnp.maximum(h + b1_ref[...], 0.0)
        z = jnp.dot(h, w2_ref[...], preferred_element_type=jnp.float32)
        o_ref[...] = jax.nn.sigmoid(z + b2_ref[...])

    return pl.pallas_call(
        body,
        grid=(BATCH // bt,),
        in_specs=[
            pl.BlockSpec((bt, EMBD), lambda i: (i, 0)),
            pl.BlockSpec((EMBD, h1), lambda i: (0, 0)),
            pl.BlockSpec((1, h1), lambda i: (0, 0)),
            pl.BlockSpec((h1, h2), lambda i: (0, 0)),
            pl.BlockSpec((1, h2), lambda i: (0, 0)),
        ],
        out_specs=pl.BlockSpec((bt, h2), lambda i: (i, 0)),
        out_shape=jax.ShapeDtypeStruct((BATCH, h2), jnp.float32),
    )(xm, w1, b1.reshape(1, -1), w2, b2.reshape(1, -1))


def kernel(II, emb, W1, b1, W2, b2):
    iit = II.T.reshape(-1)
    xm = _sc_pool(iit, emb)
    return _tc_mlp(xm, W1, b1, W2, b2)


# speedup vs baseline: 2.0036x; 1.0177x over previous
import functools

import jax
import jax.numpy as jnp
from jax import lax
from jax.experimental import pallas as pl
from jax.experimental.pallas import tpu as pltpu
from jax.experimental.pallas import tpu_sc as plsc

LSEQ = 200
EMBD = 32
BATCH = 4096
NWORK = 32
BPW = BATCH // NWORK
HALF = 16


def _sc_pool(iit, emb):
    mesh = plsc.VectorSubcoreMesh(core_axis_name="c", subcore_axis_name="s")

    @functools.partial(
        pl.kernel,
        out_type=jax.ShapeDtypeStruct((BATCH, EMBD), jnp.float32),
        mesh=mesh,
        scratch_types=[
            pltpu.VMEM((BPW * LSEQ,), jnp.int32),
            pltpu.VMEM((LSEQ, EMBD), jnp.float32),
            pltpu.VMEM((LSEQ, EMBD), jnp.float32),
            pltpu.VMEM((BPW, EMBD), jnp.float32),
            pltpu.SemaphoreType.DMA,
            pltpu.SemaphoreType.DMA,
        ],
        compiler_params=pltpu.CompilerParams(use_tc_tiling_on_sc=False),
    )
    def pool(ii_hbm, emb_hbm, out_hbm, idx_v, rows0, rows1, xm_v, sem0, sem1):
        wid = lax.axis_index("s") * 2 + lax.axis_index("c")
        base = wid * BPW
        pltpu.sync_copy(ii_hbm.at[pl.ds(base * LSEQ, BPW * LSEQ)], idx_v)

        def idx_slice(b):
            return idx_v.at[pl.ds(pl.multiple_of(b * LSEQ, 8), LSEQ)]

        def start_gather(b, rows_ref, sem):
            pltpu.async_copy(emb_hbm.at[idx_slice(b)], rows_ref, sem)

        def wait_gather(b, rows_ref, sem):
            pltpu.make_async_copy(emb_hbm.at[idx_slice(b)], rows_ref, sem).wait()

        def reduce_rows(rows_ref, b):
            zero = jnp.zeros((HALF,), jnp.float32)
            one = jnp.full((HALF,), 1.0, jnp.float32)

            def body(l, carry):
                s0, s1, c0, c1 = carry
                x0 = rows_ref[l, pl.ds(0, HALF)]
                x1 = rows_ref[l, pl.ds(HALF, HALF)]
                s0 = s0 + x0
                s1 = s1 + x1
                c0 = c0 + jnp.where(x0 != zero, one, zero)
                c1 = c1 + jnp.where(x1 != zero, one, zero)
                return s0, s1, c0, c1

            s0, s1, c0, c1 = lax.fori_loop(0, LSEQ, body, (zero, zero, zero, zero))
            xm_v[b, pl.ds(0, HALF)] = s0 / jnp.maximum(c0, one)
            xm_v[b, pl.ds(HALF, HALF)] = s1 / jnp.maximum(c1, one)

        start_gather(0, rows0, sem0)

        def outer(j, carry):
            b0 = 2 * j
            start_gather(b0 + 1, rows1, sem1)
            wait_gather(b0, rows0, sem0)
            reduce_rows(rows0, b0)

            @pl.when(j < BPW // 2 - 1)
            def _():
                start_gather(b0 + 2, rows0, sem0)

            wait_gather(b0 + 1, rows1, sem1)
            reduce_rows(rows1, b0 + 1)
            return carry

        lax.fori_loop(0, BPW // 2, outer, 0)
        pltpu.sync_copy(xm_v, out_hbm.at[pl.ds(base, BPW), :])

    return pool(iit, emb)


def _tc_mlp(xm, w1, b1, w2, b2):
    h1 = w1.shape[1]
    h2 = w2.shape[1]
    bt = 512

    def body(x_ref, w1_ref, b1_ref, w2_ref, b2_ref, o_ref):
        x = x_ref[...]
        h = jnp.dot(x, w1_ref[...], preferred_element_type=jnp.float32)
        h = jnp.maximum(h + b1_ref[...], 0.0)
        z = jnp.dot(h, w2_ref[...], preferred_element_type=jnp.float32)
        o_ref[...] = jax.nn.sigmoid(z + b2_ref[...])

    return pl.pallas_call(
        body,
        grid=(BATCH // bt,),
        in_specs=[
            pl.BlockSpec((bt, EMBD), lambda i: (i, 0)),
            pl.BlockSpec((EMBD, h1), lambda i: (0, 0)),
            pl.BlockSpec((1, h1), lambda i: (0, 0)),
            pl.BlockSpec((h1, h2), lambda i: (0, 0)),
            pl.BlockSpec((1, h2), lambda i: (0, 0)),
        ],
        out_specs=pl.BlockSpec((bt, h2), lambda i: (i, 0)),
        out_shape=jax.ShapeDtypeStruct((BATCH, h2), jnp.float32),
    )(xm, w1, b1.reshape(1, -1), w2, b2.reshape(1, -1))


def kernel(II, emb, W1, b1, W2, b2):
    iit = (II * 4).T.reshape(-1)
    nrows = emb.shape[0]
    embp = jnp.pad(emb, ((0, 0), (0, 128 - EMBD))).reshape(4 * nrows, EMBD)
    xm = _sc_pool(iit, embp)
    return _tc_mlp(xm, W1, b1, W2, b2)


# speedup vs baseline: 2.7070x; 1.3511x over previous
import functools

import jax
import jax.numpy as jnp
from jax import lax
from jax.experimental import pallas as pl
from jax.experimental.pallas import tpu as pltpu
from jax.experimental.pallas import tpu_sc as plsc

LSEQ = 200
EMBD = 32
BATCH = 4096
NWORK = 32
BPW = BATCH // NWORK
HALF = 16


def _sc_pool(iit, emb):
    mesh = plsc.VectorSubcoreMesh(core_axis_name="c", subcore_axis_name="s")

    @functools.partial(
        pl.kernel,
        out_type=jax.ShapeDtypeStruct((BATCH, EMBD), jnp.float32),
        mesh=mesh,
        scratch_types=[
            pltpu.VMEM((BPW * LSEQ,), jnp.int32),
            pltpu.VMEM((LSEQ, EMBD), jnp.float32),
            pltpu.VMEM((LSEQ, EMBD), jnp.float32),
            pltpu.VMEM((BPW, EMBD), jnp.float32),
            pltpu.SemaphoreType.DMA,
            pltpu.SemaphoreType.DMA,
        ],
        compiler_params=pltpu.CompilerParams(use_tc_tiling_on_sc=False),
    )
    def pool(ii_hbm, emb_hbm, out_hbm, idx_v, rows0, rows1, xm_v, sem0, sem1):
        wid = lax.axis_index("s") * 2 + lax.axis_index("c")
        base = wid * BPW
        pltpu.sync_copy(ii_hbm.at[pl.ds(base * LSEQ, BPW * LSEQ)], idx_v)

        def idx_slice(b):
            return idx_v.at[pl.ds(pl.multiple_of(b * LSEQ, 8), LSEQ)]

        def start_gather(b, rows_ref, sem):
            pltpu.async_copy(emb_hbm.at[idx_slice(b)], rows_ref, sem)

        def wait_gather(b, rows_ref, sem):
            pltpu.make_async_copy(emb_hbm.at[idx_slice(b)], rows_ref, sem).wait()

        def reduce_rows(rows_ref, b):
            zero = jnp.zeros((HALF,), jnp.float32)
            one = jnp.full((HALF,), 1.0, jnp.float32)

            def body(l, carry):
                s0, s1, c0, c1 = carry
                x0 = rows_ref[l, pl.ds(0, HALF)]
                x1 = rows_ref[l, pl.ds(HALF, HALF)]
                s0 = s0 + x0
                s1 = s1 + x1
                c0 = c0 + jnp.where(x0 != zero, one, zero)
                c1 = c1 + jnp.where(x1 != zero, one, zero)
                return s0, s1, c0, c1

            s0, s1, c0, c1 = lax.fori_loop(0, LSEQ, body, (zero, zero, zero, zero))
            xm_v[b, pl.ds(0, HALF)] = s0 / jnp.maximum(c0, one)
            xm_v[b, pl.ds(HALF, HALF)] = s1 / jnp.maximum(c1, one)

        start_gather(0, rows0, sem0)

        def outer(j, carry):
            b0 = 2 * j
            start_gather(b0 + 1, rows1, sem1)
            wait_gather(b0, rows0, sem0)
            reduce_rows(rows0, b0)

            @pl.when(j < BPW // 2 - 1)
            def _():
                start_gather(b0 + 2, rows0, sem0)

            wait_gather(b0 + 1, rows1, sem1)
            reduce_rows(rows1, b0 + 1)
            return carry

        lax.fori_loop(0, BPW // 2, outer, 0)
        pltpu.sync_copy(xm_v, out_hbm.at[pl.ds(base, BPW), :])

    return pool(iit, emb)


def _tc_table_rows(embt, nrows):
    bk = 4096
    m = ((nrows + bk - 1) // bk) * bk

    def body(x_ref, o_ref):
        xt = x_ref[...].T
        o_ref[...] = jnp.concatenate(
            [xt, jnp.zeros((bk, 3 * EMBD), jnp.float32)], axis=1)

    return pl.pallas_call(
        body,
        grid=(m // bk,),
        in_specs=[pl.BlockSpec((EMBD, bk), lambda i: (0, i))],
        out_specs=pl.BlockSpec((bk, 4 * EMBD), lambda i: (i, 0)),
        out_shape=jax.ShapeDtypeStruct((m, 4 * EMBD), jnp.float32),
    )(embt)


def _tc_mlp(xm, w1, b1, w2, b2):
    h1 = w1.shape[1]
    h2 = w2.shape[1]
    bt = 512

    def body(x_ref, w1_ref, b1_ref, w2_ref, b2_ref, o_ref):
        x = x_ref[...]
        h = jnp.dot(x, w1_ref[...], preferred_element_type=jnp.float32)
        h = jnp.maximum(h + b1_ref[...], 0.0)
        z = jnp.dot(h, w2_ref[...], preferred_element_type=jnp.float32)
        o_ref[...] = jax.nn.sigmoid(z + b2_ref[...])

    return pl.pallas_call(
        body,
        grid=(BATCH // bt,),
        in_specs=[
            pl.BlockSpec((bt, EMBD), lambda i: (i, 0)),
            pl.BlockSpec((EMBD, h1), lambda i: (0, 0)),
            pl.BlockSpec((1, h1), lambda i: (0, 0)),
            pl.BlockSpec((h1, h2), lambda i: (0, 0)),
            pl.BlockSpec((1, h2), lambda i: (0, 0)),
        ],
        out_specs=pl.BlockSpec((bt, h2), lambda i: (i, 0)),
        out_shape=jax.ShapeDtypeStruct((BATCH, h2), jnp.float32),
    )(xm, w1, b1.reshape(1, -1), w2, b2.reshape(1, -1))


def kernel(II, emb, W1, b1, W2, b2):
    iit = (II * 4).T.reshape(-1)
    nrows = emb.shape[0]
    embp = _tc_table_rows(emb.T, nrows)
    table = embp.reshape(embp.shape[0] * 4, EMBD)
    xm = _sc_pool(iit, table)
    return _tc_mlp(xm, W1, b1, W2, b2)


# speedup vs baseline: 3.0443x; 1.1246x over previous
import functools

import jax
import jax.numpy as jnp
from jax import lax
from jax.experimental import pallas as pl
from jax.experimental.pallas import tpu as pltpu
from jax.experimental.pallas import tpu_sc as plsc

LSEQ = 200
EMBD = 32
BATCH = 4096
NWORK = 32
BPW = BATCH // NWORK
HALF = 16
GRP = 4


def _sc_pool(iit, emb):
    mesh = plsc.VectorSubcoreMesh(core_axis_name="c", subcore_axis_name="s")

    @functools.partial(
        pl.kernel,
        out_type=jax.ShapeDtypeStruct((BATCH, EMBD), jnp.float32),
        mesh=mesh,
        scratch_types=[
            pltpu.VMEM((BPW * LSEQ,), jnp.int32),
            pltpu.VMEM((GRP * LSEQ, EMBD), jnp.float32),
            pltpu.VMEM((GRP * LSEQ, EMBD), jnp.float32),
            pltpu.VMEM((BPW, EMBD), jnp.float32),
            pltpu.SemaphoreType.DMA,
            pltpu.SemaphoreType.DMA,
        ],
        compiler_params=pltpu.CompilerParams(use_tc_tiling_on_sc=False),
    )
    def pool(ii_hbm, emb_hbm, out_hbm, idx_v, rows0, rows1, xm_v, sem0, sem1):
        wid = lax.axis_index("s") * 2 + lax.axis_index("c")
        base = wid * BPW
        pltpu.sync_copy(ii_hbm.at[pl.ds(base * LSEQ, BPW * LSEQ)], idx_v)

        def idx_slice(g):
            return idx_v.at[pl.ds(pl.multiple_of(g * (GRP * LSEQ), 8),
                                  GRP * LSEQ)]

        def start_gather(g, rows_ref, sem):
            pltpu.async_copy(emb_hbm.at[idx_slice(g)], rows_ref, sem)

        def wait_gather(g, rows_ref, sem):
            pltpu.make_async_copy(emb_hbm.at[idx_slice(g)], rows_ref, sem).wait()

        zero = jnp.zeros((HALF,), jnp.float32)
        one = jnp.full((HALF,), 1.0, jnp.float32)
        lseq = jnp.full((HALF,), float(LSEQ), jnp.float32)

        def reduce_rows(rows_ref, koff, b):
            def body(i, carry):
                s0, s1, z0, z1 = carry
                l = koff + 2 * i
                xa0 = rows_ref[l, pl.ds(0, HALF)]
                xa1 = rows_ref[l, pl.ds(HALF, HALF)]
                xb0 = rows_ref[l + 1, pl.ds(0, HALF)]
                xb1 = rows_ref[l + 1, pl.ds(HALF, HALF)]
                s0 = s0 + xa0 + xb0
                s1 = s1 + xa1 + xb1
                z0 = z0 + jnp.where(xa0 == zero, one, zero)
                z0 = z0 + jnp.where(xb0 == zero, one, zero)
                z1 = z1 + jnp.where(xa1 == zero, one, zero)
                z1 = z1 + jnp.where(xb1 == zero, one, zero)
                return s0, s1, z0, z1

            s0, s1, z0, z1 = lax.fori_loop(0, LSEQ // 2, body,
                                           (zero, zero, zero, zero))
            xm_v[b, pl.ds(0, HALF)] = s0 / jnp.maximum(lseq - z0, one)
            xm_v[b, pl.ds(HALF, HALF)] = s1 / jnp.maximum(lseq - z1, one)

        def reduce_group(rows_ref, g):
            for k in range(GRP):
                reduce_rows(rows_ref, k * LSEQ, g * GRP + k)

        ngrp = BPW // GRP
        start_gather(0, rows0, sem0)

        def outer(j, carry):
            g0 = 2 * j
            start_gather(g0 + 1, rows1, sem1)
            wait_gather(g0, rows0, sem0)
            reduce_group(rows0, g0)

            @pl.when(j < ngrp // 2 - 1)
            def _():
                start_gather(g0 + 2, rows0, sem0)

            wait_gather(g0 + 1, rows1, sem1)
            reduce_group(rows1, g0 + 1)
            return carry

        lax.fori_loop(0, ngrp // 2, outer, 0)
        pltpu.sync_copy(xm_v, out_hbm.at[pl.ds(base, BPW), :])

    return pool(iit, emb)


def _tc_table_rows(embt, nrows):
    bk = 4096
    m = ((nrows + bk - 1) // bk) * bk

    def body(x_ref, o_ref):
        xt = x_ref[...].T
        o_ref[...] = jnp.concatenate(
            [xt, jnp.zeros((bk, 3 * EMBD), jnp.float32)], axis=1)

    return pl.pallas_call(
        body,
        grid=(m // bk,),
        in_specs=[pl.BlockSpec((EMBD, bk), lambda i: (0, i))],
        out_specs=pl.BlockSpec((bk, 4 * EMBD), lambda i: (i, 0)),
        out_shape=jax.ShapeDtypeStruct((m, 4 * EMBD), jnp.float32),
    )(embt)


def _tc_mlp(xm, w1, b1, w2, b2):
    h1 = w1.shape[1]
    h2 = w2.shape[1]
    bt = 512

    def body(x_ref, w1_ref, b1_ref, w2_ref, b2_ref, o_ref):
        x = x_ref[...]
        h = jnp.dot(x, w1_ref[...], preferred_element_type=jnp.float32)
        h = jnp.maximum(h + b1_ref[...], 0.0)
        z = jnp.dot(h, w2_ref[...], preferred_element_type=jnp.float32)
        o_ref[...] = jax.nn.sigmoid(z + b2_ref[...])

    return pl.pallas_call(
        body,
        grid=(BATCH // bt,),
        in_specs=[
            pl.BlockSpec((bt, EMBD), lambda i: (i, 0)),
            pl.BlockSpec((EMBD, h1), lambda i: (0, 0)),
            pl.BlockSpec((1, h1), lambda i: (0, 0)),
            pl.BlockSpec((h1, h2), lambda i: (0, 0)),
            pl.BlockSpec((1, h2), lambda i: (0, 0)),
        ],
        out_specs=pl.BlockSpec((bt, h2), lambda i: (i, 0)),
        out_shape=jax.ShapeDtypeStruct((BATCH, h2), jnp.float32),
    )(xm, w1, b1.reshape(1, -1), w2, b2.reshape(1, -1))


def kernel(II, emb, W1, b1, W2, b2):
    iit = (II * 4).T.reshape(-1)
    nrows = emb.shape[0]
    embp = _tc_table_rows(emb.T, nrows)
    table = embp.reshape(embp.shape[0] * 4, EMBD)
    xm = _sc_pool(iit, table)
    return _tc_mlp(xm, W1, b1, W2, b2)


# speedup vs baseline: 3.6771x; 1.2079x over previous
import functools

import jax
import jax.numpy as jnp
from jax import lax
from jax.experimental import pallas as pl
from jax.experimental.pallas import tpu as pltpu
from jax.experimental.pallas import tpu_sc as plsc

LSEQ = 200
EMBD = 32
BATCH = 4096
NWORK = 32
BPW = BATCH // NWORK
HALF = 16
GRP = 4


def _sc_pool(iit, emb):
    mesh = plsc.VectorSubcoreMesh(core_axis_name="c", subcore_axis_name="s")

    @functools.partial(
        pl.kernel,
        out_type=jax.ShapeDtypeStruct((BATCH, EMBD), jnp.float32),
        mesh=mesh,
        scratch_types=[
            pltpu.VMEM((BPW * LSEQ,), jnp.int32),
            pltpu.VMEM((GRP * LSEQ, EMBD), jnp.float32),
            pltpu.VMEM((GRP * LSEQ, EMBD), jnp.float32),
            pltpu.VMEM((BPW, EMBD), jnp.float32),
            pltpu.SemaphoreType.DMA,
            pltpu.SemaphoreType.DMA,
        ],
        compiler_params=pltpu.CompilerParams(use_tc_tiling_on_sc=False),
    )
    def pool(ii_hbm, emb_hbm, out_hbm, idx_v, rows0, rows1, xm_v, sem0, sem1):
        wid = lax.axis_index("s") * 2 + lax.axis_index("c")
        base = wid * BPW
        pltpu.sync_copy(ii_hbm.at[pl.ds(base * LSEQ, BPW * LSEQ)], idx_v)

        def idx_slice(g):
            return idx_v.at[pl.ds(pl.multiple_of(g * (GRP * LSEQ), 8),
                                  GRP * LSEQ)]

        def start_gather(g, rows_ref, sem):
            pltpu.async_copy(emb_hbm.at[idx_slice(g)], rows_ref, sem)

        def wait_gather(g, rows_ref, sem):
            pltpu.make_async_copy(emb_hbm.at[idx_slice(g)], rows_ref, sem).wait()

        zero = jnp.zeros((HALF,), jnp.float32)
        one = jnp.full((HALF,), 1.0, jnp.float32)
        lseq = jnp.full((HALF,), float(LSEQ), jnp.float32)

        def reduce_rows(rows_ref, koff, b):
            def body(i, carry):
                s0, s1, z0, z1 = carry
                l = koff + 2 * i
                xa0 = rows_ref[l, pl.ds(0, HALF)]
                xa1 = rows_ref[l, pl.ds(HALF, HALF)]
                xb0 = rows_ref[l + 1, pl.ds(0, HALF)]
                xb1 = rows_ref[l + 1, pl.ds(HALF, HALF)]
                s0 = s0 + xa0 + xb0
                s1 = s1 + xa1 + xb1
                z0 = z0 + jnp.where(xa0 == zero, one, zero)
                z0 = z0 + jnp.where(xb0 == zero, one, zero)
                z1 = z1 + jnp.where(xa1 == zero, one, zero)
                z1 = z1 + jnp.where(xb1 == zero, one, zero)
                return s0, s1, z0, z1

            s0, s1, z0, z1 = lax.fori_loop(0, LSEQ // 2, body,
                                           (zero, zero, zero, zero))
            xm_v[b, pl.ds(0, HALF)] = s0 / jnp.maximum(lseq - z0, one)
            xm_v[b, pl.ds(HALF, HALF)] = s1 / jnp.maximum(lseq - z1, one)

        def reduce_group(rows_ref, g):
            for k in range(GRP):
                reduce_rows(rows_ref, k * LSEQ, g * GRP + k)

        ngrp = BPW // GRP
        start_gather(0, rows0, sem0)

        def outer(j, carry):
            g0 = 2 * j
            start_gather(g0 + 1, rows1, sem1)
            wait_gather(g0, rows0, sem0)
            reduce_group(rows0, g0)

            @pl.when(j < ngrp // 2 - 1)
            def _():
                start_gather(g0 + 2, rows0, sem0)

            wait_gather(g0 + 1, rows1, sem1)
            reduce_group(rows1, g0 + 1)
            return carry

        lax.fori_loop(0, ngrp // 2, outer, 0)
        pltpu.sync_copy(xm_v, out_hbm.at[pl.ds(base, BPW), :])

    return pool(iit, emb)


def _tc_table_rows(embt, nrows):
    bk = 8192
    m = ((nrows + bk - 1) // bk) * bk

    def body(x_ref, o_ref):
        eye = jnp.eye(EMBD, EMBD, dtype=jnp.float32)
        o_ref[:, 0:EMBD] = jax.lax.dot_general(
            x_ref[...], eye, (((0,), (0,)), ((), ())),
            preferred_element_type=jnp.float32)

    return pl.pallas_call(
        body,
        grid=(m // bk,),
        in_specs=[pl.BlockSpec((EMBD, bk), lambda i: (0, i))],
        out_specs=pl.BlockSpec((bk, 4 * EMBD), lambda i: (i, 0)),
        out_shape=jax.ShapeDtypeStruct((m, 4 * EMBD), jnp.float32),
        compiler_params=pltpu.CompilerParams(
            fuse_transposed_lhs_in_matmul=True),
    )(embt)


def _tc_mlp(xm, w1, b1, w2, b2):
    h1 = w1.shape[1]
    h2 = w2.shape[1]
    bt = 512

    def body(x_ref, w1_ref, b1_ref, w2_ref, b2_ref, o_ref):
        x = x_ref[...]
        h = jnp.dot(x, w1_ref[...], preferred_element_type=jnp.float32)
        h = jnp.maximum(h + b1_ref[...], 0.0)
        z = jnp.dot(h, w2_ref[...], preferred_element_type=jnp.float32)
        o_ref[...] = jax.nn.sigmoid(z + b2_ref[...])

    return pl.pallas_call(
        body,
        grid=(BATCH // bt,),
        in_specs=[
            pl.BlockSpec((bt, EMBD), lambda i: (i, 0)),
            pl.BlockSpec((EMBD, h1), lambda i: (0, 0)),
            pl.BlockSpec((1, h1), lambda i: (0, 0)),
            pl.BlockSpec((h1, h2), lambda i: (0, 0)),
            pl.BlockSpec((1, h2), lambda i: (0, 0)),
        ],
        out_specs=pl.BlockSpec((bt, h2), lambda i: (i, 0)),
        out_shape=jax.ShapeDtypeStruct((BATCH, h2), jnp.float32),
    )(xm, w1, b1.reshape(1, -1), w2, b2.reshape(1, -1))


def kernel(II, emb, W1, b1, W2, b2):
    iit = (II * 4).T.reshape(-1)
    nrows = emb.shape[0]
    embp = _tc_table_rows(emb.T, nrows)
    table = embp.reshape(embp.shape[0] * 4, EMBD)
    xm = _sc_pool(iit, table)
    return _tc_mlp(xm, W1, b1, W2, b2)


# speedup vs baseline: 4.0376x; 1.0980x over previous
import functools

import jax
import jax.numpy as jnp
from jax import lax
from jax.experimental import pallas as pl
from jax.experimental.pallas import tpu as pltpu
from jax.experimental.pallas import tpu_sc as plsc

LSEQ = 200
EMBD = 32
BATCH = 4096
NWORK = 32
BPW = BATCH // NWORK
HALF = 16
GRP = 4


def _sc_pool(iit, emb):
    mesh = plsc.VectorSubcoreMesh(core_axis_name="c", subcore_axis_name="s")

    @functools.partial(
        pl.kernel,
        out_type=jax.ShapeDtypeStruct((BATCH, EMBD), jnp.float32),
        mesh=mesh,
        scratch_types=[
            pltpu.VMEM((BPW * LSEQ,), jnp.int32),
            pltpu.VMEM((GRP * LSEQ, EMBD), jnp.float32),
            pltpu.VMEM((GRP * LSEQ, EMBD), jnp.float32),
            pltpu.VMEM((BPW, EMBD), jnp.float32),
            pltpu.SemaphoreType.DMA,
            pltpu.SemaphoreType.DMA,
        ],
        compiler_params=pltpu.CompilerParams(use_tc_tiling_on_sc=False),
    )
    def pool(ii_hbm, emb_hbm, out_hbm, idx_v, rows0, rows1, xm_v, sem0, sem1):
        wid = lax.axis_index("s") * 2 + lax.axis_index("c")
        base = wid * BPW
        pltpu.sync_copy(ii_hbm.at[pl.ds(base * LSEQ, BPW * LSEQ)], idx_v)

        def idx_slice(g):
            return idx_v.at[pl.ds(pl.multiple_of(g * (GRP * LSEQ), 8),
                                  GRP * LSEQ)]

        def start_gather(g, rows_ref, sem):
            pltpu.async_copy(emb_hbm.at[idx_slice(g)], rows_ref, sem)

        def wait_gather(g, rows_ref, sem):
            pltpu.make_async_copy(emb_hbm.at[idx_slice(g)], rows_ref, sem).wait()

        zero = jnp.zeros((HALF,), jnp.float32)
        one = jnp.full((HALF,), 1.0, jnp.float32)
        lseq = jnp.full((HALF,), float(LSEQ), jnp.float32)

        def reduce_rows(rows_ref, koff, b):
            def body(i, carry):
                s0, s1, z0, z1 = carry
                l = koff + 2 * i
                xa0 = rows_ref[l, pl.ds(0, HALF)]
                xa1 = rows_ref[l, pl.ds(HALF, HALF)]
                xb0 = rows_ref[l + 1, pl.ds(0, HALF)]
                xb1 = rows_ref[l + 1, pl.ds(HALF, HALF)]
                s0 = s0 + xa0 + xb0
                s1 = s1 + xa1 + xb1
                z0 = z0 + jnp.where(xa0 == zero, one, zero)
                z0 = z0 + jnp.where(xb0 == zero, one, zero)
                z1 = z1 + jnp.where(xa1 == zero, one, zero)
                z1 = z1 + jnp.where(xb1 == zero, one, zero)
                return s0, s1, z0, z1

            s0, s1, z0, z1 = lax.fori_loop(0, LSEQ // 2, body,
                                           (zero, zero, zero, zero))
            xm_v[b, pl.ds(0, HALF)] = s0 / jnp.maximum(lseq - z0, one)
            xm_v[b, pl.ds(HALF, HALF)] = s1 / jnp.maximum(lseq - z1, one)

        def reduce_group(rows_ref, g):
            for k in range(GRP):
                reduce_rows(rows_ref, k * LSEQ, g * GRP + k)

        ngrp = BPW // GRP
        start_gather(0, rows0, sem0)

        def outer(j, carry):
            g0 = 2 * j
            start_gather(g0 + 1, rows1, sem1)
            wait_gather(g0, rows0, sem0)
            reduce_group(rows0, g0)

            @pl.when(j < ngrp // 2 - 1)
            def _():
                start_gather(g0 + 2, rows0, sem0)

            wait_gather(g0 + 1, rows1, sem1)
            reduce_group(rows1, g0 + 1)
            return carry

        lax.fori_loop(0, ngrp // 2, outer, 0)
        pltpu.sync_copy(xm_v, out_hbm.at[pl.ds(base, BPW), :])

    return pool(iit, emb)


def _tc_table_rows(embt, nrows):
    bk = 16384
    m = ((nrows + bk - 1) // bk) * bk

    def body(x_ref, o_ref):
        o_ref[:, 0:EMBD] = x_ref[...].T

    return pl.pallas_call(
        body,
        grid=(m // bk,),
        in_specs=[pl.BlockSpec((EMBD, bk), lambda i: (0, i))],
        out_specs=pl.BlockSpec((bk, 4 * EMBD), lambda i: (i, 0)),
        out_shape=jax.ShapeDtypeStruct((m, 4 * EMBD), jnp.float32),
    )(embt)


def _tc_mlp(xm, w1, b1, w2, b2):
    h1 = w1.shape[1]
    h2 = w2.shape[1]
    bt = 512

    def body(x_ref, w1_ref, b1_ref, w2_ref, b2_ref, o_ref):
        x = x_ref[...]
        h = jnp.dot(x, w1_ref[...], preferred_element_type=jnp.float32)
        h = jnp.maximum(h + b1_ref[...], 0.0)
        z = jnp.dot(h, w2_ref[...], preferred_element_type=jnp.float32)
        o_ref[...] = jax.nn.sigmoid(z + b2_ref[...])

    return pl.pallas_call(
        body,
        grid=(BATCH // bt,),
        in_specs=[
            pl.BlockSpec((bt, EMBD), lambda i: (i, 0)),
            pl.BlockSpec((EMBD, h1), lambda i: (0, 0)),
            pl.BlockSpec((1, h1), lambda i: (0, 0)),
            pl.BlockSpec((h1, h2), lambda i: (0, 0)),
            pl.BlockSpec((1, h2), lambda i: (0, 0)),
        ],
        out_specs=pl.BlockSpec((bt, h2), lambda i: (i, 0)),
        out_shape=jax.ShapeDtypeStruct((BATCH, h2), jnp.float32),
    )(xm, w1, b1.reshape(1, -1), w2, b2.reshape(1, -1))


def kernel(II, emb, W1, b1, W2, b2):
    iit = (II * 4).T.reshape(-1)
    nrows = emb.shape[0]
    embp = _tc_table_rows(emb.T, nrows)
    table = embp.reshape(embp.shape[0] * 4, EMBD)
    xm = _sc_pool(iit, table)
    return _tc_mlp(xm, W1, b1, W2, b2)


# speedup vs baseline: 4.0786x; 1.0102x over previous
import functools

import jax
import jax.numpy as jnp
from jax import lax
from jax.experimental import pallas as pl
from jax.experimental.pallas import tpu as pltpu
from jax.experimental.pallas import tpu_sc as plsc

LSEQ = 200
EMBD = 32
BATCH = 4096
NWORK = 32
BPW = BATCH // NWORK
HALF = 16
GRP = 4


def _sc_pool(iit, emb):
    mesh = plsc.VectorSubcoreMesh(core_axis_name="c", subcore_axis_name="s")

    @functools.partial(
        pl.kernel,
        out_type=jax.ShapeDtypeStruct((BATCH, EMBD), jnp.float32),
        mesh=mesh,
        scratch_types=[
            pltpu.VMEM((BPW * LSEQ,), jnp.int32),
            pltpu.VMEM((GRP * LSEQ, EMBD), jnp.float32),
            pltpu.VMEM((GRP * LSEQ, EMBD), jnp.float32),
            pltpu.VMEM((BPW, EMBD), jnp.float32),
            pltpu.SemaphoreType.DMA,
            pltpu.SemaphoreType.DMA,
        ],
        compiler_params=pltpu.CompilerParams(use_tc_tiling_on_sc=False),
    )
    def pool(ii_hbm, emb_hbm, out_hbm, idx_v, rows0, rows1, xm_v, sem0, sem1):
        wid = lax.axis_index("s") * 2 + lax.axis_index("c")
        base = wid * BPW
        pltpu.sync_copy(ii_hbm.at[pl.ds(base * LSEQ, BPW * LSEQ)], idx_v)

        def idx_slice(g):
            return idx_v.at[pl.ds(pl.multiple_of(g * (GRP * LSEQ), 8),
                                  GRP * LSEQ)]

        def start_gather(g, rows_ref, sem):
            pltpu.async_copy(emb_hbm.at[idx_slice(g)], rows_ref, sem)

        def wait_gather(g, rows_ref, sem):
            pltpu.make_async_copy(emb_hbm.at[idx_slice(g)], rows_ref, sem).wait()

        zero = jnp.zeros((HALF,), jnp.float32)
        one = jnp.full((HALF,), 1.0, jnp.float32)
        lseq = jnp.full((HALF,), float(LSEQ), jnp.float32)

        def reduce_rows(rows_ref, koff, b):
            def body(i, carry):
                s0, s1, z0, z1 = carry
                l = koff + 4 * i
                for u in range(4):
                    x0 = rows_ref[l + u, pl.ds(0, HALF)]
                    x1 = rows_ref[l + u, pl.ds(HALF, HALF)]
                    s0 = s0 + x0
                    s1 = s1 + x1
                    z0 = z0 + jnp.where(x0 == zero, one, zero)
                    z1 = z1 + jnp.where(x1 == zero, one, zero)
                return s0, s1, z0, z1

            s0, s1, z0, z1 = lax.fori_loop(0, LSEQ // 4, body,
                                           (zero, zero, zero, zero))
            xm_v[b, pl.ds(0, HALF)] = s0 / jnp.maximum(lseq - z0, one)
            xm_v[b, pl.ds(HALF, HALF)] = s1 / jnp.maximum(lseq - z1, one)

        def reduce_group(rows_ref, g):
            for k in range(GRP):
                reduce_rows(rows_ref, k * LSEQ, g * GRP + k)

        ngrp = BPW // GRP
        start_gather(0, rows0, sem0)

        def outer(j, carry):
            g0 = 2 * j
            start_gather(g0 + 1, rows1, sem1)
            wait_gather(g0, rows0, sem0)
            reduce_group(rows0, g0)

            @pl.when(j < ngrp // 2 - 1)
            def _():
                start_gather(g0 + 2, rows0, sem0)

            wait_gather(g0 + 1, rows1, sem1)
            reduce_group(rows1, g0 + 1)
            return carry

        lax.fori_loop(0, ngrp // 2, outer, 0)
        pltpu.sync_copy(xm_v, out_hbm.at[pl.ds(base, BPW), :])

    return pool(iit, emb)


def _tc_table_rows(embt, nrows):
    bk = 32768
    m = ((nrows + bk - 1) // bk) * bk

    def body(x_ref, o_ref):
        o_ref[:, 0:EMBD] = x_ref[...].T

    return pl.pallas_call(
        body,
        grid=(m // bk,),
        in_specs=[pl.BlockSpec((EMBD, bk), lambda i: (0, i))],
        out_specs=pl.BlockSpec((bk, 4 * EMBD), lambda i: (i, 0)),
        out_shape=jax.ShapeDtypeStruct((m, 4 * EMBD), jnp.float32),
    )(embt)


def _tc_mlp(xm, w1, b1, w2, b2):
    h1 = w1.shape[1]
    h2 = w2.shape[1]
    bt = 512

    def body(x_ref, w1_ref, b1_ref, w2_ref, b2_ref, o_ref):
        x = x_ref[...]
        h = jnp.dot(x, w1_ref[...], preferred_element_type=jnp.float32)
        h = jnp.maximum(h + b1_ref[...], 0.0)
        z = jnp.dot(h, w2_ref[...], preferred_element_type=jnp.float32)
        o_ref[...] = jax.nn.sigmoid(z + b2_ref[...])

    return pl.pallas_call(
        body,
        grid=(BATCH // bt,),
        in_specs=[
            pl.BlockSpec((bt, EMBD), lambda i: (i, 0)),
            pl.BlockSpec((EMBD, h1), lambda i: (0, 0)),
            pl.BlockSpec((1, h1), lambda i: (0, 0)),
            pl.BlockSpec((h1, h2), lambda i: (0, 0)),
            pl.BlockSpec((1, h2), lambda i: (0, 0)),
        ],
        out_specs=pl.BlockSpec((bt, h2), lambda i: (i, 0)),
        out_shape=jax.ShapeDtypeStruct((BATCH, h2), jnp.float32),
    )(xm, w1, b1.reshape(1, -1), w2, b2.reshape(1, -1))


def kernel(II, emb, W1, b1, W2, b2):
    iit = (II * 4).T.reshape(-1)
    nrows = emb.shape[0]
    embp = _tc_table_rows(emb.T, nrows)
    table = embp.reshape(embp.shape[0] * 4, EMBD)
    xm = _sc_pool(iit, table)
    return _tc_mlp(xm, W1, b1, W2, b2)
